# baseline (speedup 1.0000x reference)
import jax
import jax.numpy as jnp
from jax.experimental import pallas as pl
from jax.experimental.pallas import tpu as pltpu
from jax.experimental.pallas import tpu_sc as plsc

VOCAB = 100000
D = 128
N = 2048
TILE_V = 1024
NT = (VOCAB + TILE_V - 1) // TILE_V
GATHER_WINDOW = 128


def _emb_gather(emb, tokens):
    mesh = plsc.VectorSubcoreMesh(core_axis_name="core",
                                  subcore_axis_name="subcore")

    @pl.kernel(out_type=jax.ShapeDtypeStruct((N, D), emb.dtype), mesh=mesh)
    def gather_kernel(emb_hbm, idx_hbm, out_hbm):
        def body(idx_vmem, out_vmem):
            pltpu.sync_copy(emb_hbm.at[idx_vmem.at[0]], out_vmem)

        pltpu.emit_pipeline(
            body,
            grid=(N // GATHER_WINDOW,),
            in_specs=[pl.BlockSpec((1, GATHER_WINDOW),
                                   index_map=lambda i: (0, i))],
            out_specs=[pl.BlockSpec((GATHER_WINDOW, D),
                                    index_map=lambda i: (i, 0))],
            core_axis_name="subcore",
            dimension_semantics=(pltpu.PARALLEL,),
        )(idx_hbm, out_hbm)

    return gather_kernel(emb, tokens)


def _loss_body(h0_ref, w_ref, wout_ref, wlab_ref, bias_ref, out_ref,
               h_scr, s_scr):
    i = pl.program_id(0)

    @pl.when(i == 0)
    def _():
        h_scr[...] = (jnp.tanh(
            jnp.dot(h0_ref[...], w_ref[...],
                    preferred_element_type=jnp.float32)) * 8.0
                      ).astype(jnp.float8_e4m3fn)
        s_scr[...] = jnp.zeros((N, D), jnp.float32)

    lg = jnp.dot(h_scr[...],
                 (wout_ref[...] * 64.0).astype(jnp.float8_e4m3fn),
                 preferred_element_type=jnp.float32
                 ).astype(jnp.bfloat16)

    c_exp = jnp.asarray(1.4426950408889634 / 512.0, jnp.bfloat16)

    e = jnp.exp2((lg + bias_ref[0]) * c_exp)
    s_part = e[:, 0:D]
    for g in range(1, TILE_V // D):
        s_part = s_part + e[:, g * D:(g + 1) * D]
    s_scr[...] += s_part.astype(jnp.float32)

    @pl.when(i == NT - 1)
    def _():
        pk = jnp.sum(h_scr[...].astype(jnp.float32) * wlab_ref[...],
                     axis=1, keepdims=True) * (1.0 / 8.0)
        s_row = jnp.sum(s_scr[...], axis=1, keepdims=True)
        nll = jnp.log(s_row) - pk
        rows = jax.lax.broadcasted_iota(jnp.int32, (N, 1), 0)
        nll = jnp.where(rows < N - 1, nll, 0.0)
        out_ref[...] = (jnp.sum(nll) / (N - 1)).reshape(1, 1)


def kernel(x, emb, W, Wout):
    h0 = _emb_gather(emb, x)
    labels = jnp.concatenate(
        [x[0, 1:], jnp.zeros((1,), jnp.int32)]).reshape(1, N)
    wlab = _emb_gather(Wout.T, labels)
    col_ok = jnp.arange(NT * TILE_V, dtype=jnp.int32) < VOCAB
    bias = jnp.where(col_ok, 0.0, -1e30).astype(jnp.bfloat16)
    bias = bias.reshape(NT, 1, TILE_V)

    out = pl.pallas_call(
        _loss_body,
        grid=(NT,),
        in_specs=[
            pl.BlockSpec((N, D), lambda i: (0, 0)),
            pl.BlockSpec((D, D), lambda i: (0, 0)),
            pl.BlockSpec((D, TILE_V), lambda i: (0, i)),
            pl.BlockSpec((N, D), lambda i: (0, 0)),
            pl.BlockSpec((1, 1, TILE_V), lambda i: (i, 0, 0)),
        ],
        out_specs=pl.BlockSpec((1, 1), lambda i: (0, 0)),
        out_shape=jax.ShapeDtypeStruct((1, 1), jnp.float32),
        scratch_shapes=[
            pltpu.VMEM((N, D), jnp.float8_e4m3fn),
            pltpu.VMEM((N, D), jnp.float32),
        ],
    )(h0, W, Wout, wlab, bias)
    return out[0, 0]


# speedup vs baseline: 1.0499x; 1.0499x over previous
import jax
import jax.numpy as jnp
from jax.experimental import pallas as pl
from jax.experimental.pallas import tpu as pltpu
from jax.experimental.pallas import tpu_sc as plsc

VOCAB = 100000
D = 128
N = 2048
TILE_V = 2048
NT = (VOCAB + TILE_V - 1) // TILE_V
GATHER_WINDOW = 128


def _emb_gather(emb, tokens):
    mesh = plsc.VectorSubcoreMesh(core_axis_name="core",
                                  subcore_axis_name="subcore")

    @pl.kernel(out_type=jax.ShapeDtypeStruct((N, D), emb.dtype), mesh=mesh)
    def gather_kernel(emb_hbm, idx_hbm, out_hbm):
        def body(idx_vmem, out_vmem):
            pltpu.sync_copy(emb_hbm.at[idx_vmem.at[0]], out_vmem)

        pltpu.emit_pipeline(
            body,
            grid=(N // GATHER_WINDOW,),
            in_specs=[pl.BlockSpec((1, GATHER_WINDOW),
                                   index_map=lambda i: (0, i))],
            out_specs=[pl.BlockSpec((GATHER_WINDOW, D),
                                    index_map=lambda i: (i, 0))],
            core_axis_name="subcore",
            dimension_semantics=(pltpu.PARALLEL,),
        )(idx_hbm, out_hbm)

    return gather_kernel(emb, tokens)


def _loss_body(h0_ref, w_ref, wout_ref, wlab_ref, bias_ref, out_ref,
               h_scr, s_scr):
    i = pl.program_id(0)

    @pl.when(i == 0)
    def _():
        h_scr[...] = (jnp.tanh(
            jnp.dot(h0_ref[...], w_ref[...],
                    preferred_element_type=jnp.float32)) * 8.0
                      ).astype(jnp.float8_e4m3fn)
        s_scr[...] = jnp.zeros((N, D), jnp.float32)

    lg = jnp.dot(h_scr[...],
                 (wout_ref[...] * 64.0).astype(jnp.float8_e4m3fn),
                 preferred_element_type=jnp.float32
                 ).astype(jnp.bfloat16)

    c_exp = jnp.asarray(1.4426950408889634 / 512.0, jnp.bfloat16)

    e = jnp.exp2(lg * c_exp + bias_ref[0])
    s_part = e[:, 0:D]
    for g in range(1, TILE_V // D):
        s_part = s_part + e[:, g * D:(g + 1) * D]
    s_scr[...] += s_part.astype(jnp.float32)

    @pl.when(i == NT - 1)
    def _():
        pk = jnp.sum(h_scr[...].astype(jnp.float32) * wlab_ref[...],
                     axis=1, keepdims=True) * (1.0 / 8.0)
        s_row = jnp.sum(s_scr[...], axis=1, keepdims=True)
        nll = jnp.log(s_row) - pk
        rows = jax.lax.broadcasted_iota(jnp.int32, (N, 1), 0)
        nll = jnp.where(rows < N - 1, nll, 0.0)
        out_ref[...] = (jnp.sum(nll) / (N - 1)).reshape(1, 1)


def kernel(x, emb, W, Wout):
    h0 = _emb_gather(emb, x)
    labels = jnp.concatenate(
        [x[0, 1:], jnp.zeros((1,), jnp.int32)]).reshape(1, N)
    wlab = _emb_gather(Wout.T, labels)
    col_ok = jnp.arange(NT * TILE_V, dtype=jnp.int32) < VOCAB
    bias = jnp.where(col_ok, 0.0, -1e38).astype(jnp.bfloat16)
    bias = bias.reshape(NT, 1, TILE_V)

    out = pl.pallas_call(
        _loss_body,
        grid=(NT,),
        in_specs=[
            pl.BlockSpec((N, D), lambda i: (0, 0)),
            pl.BlockSpec((D, D), lambda i: (0, 0)),
            pl.BlockSpec((D, TILE_V), lambda i: (0, i)),
            pl.BlockSpec((N, D), lambda i: (0, 0)),
            pl.BlockSpec((1, 1, TILE_V), lambda i: (i, 0, 0)),
        ],
        out_specs=pl.BlockSpec((1, 1), lambda i: (0, 0)),
        out_shape=jax.ShapeDtypeStruct((1, 1), jnp.float32),
        scratch_shapes=[
            pltpu.VMEM((N, D), jnp.float8_e4m3fn),
            pltpu.VMEM((N, D), jnp.float32),
        ],
    )(h0, W, Wout, wlab, bias)
    return out[0, 0]
